# baseline (speedup 1.0000x reference)
import functools

import jax
import jax.numpy as jnp
from jax import lax
from jax.experimental import pallas as pl
from jax.experimental.pallas import tpu as pltpu
from jax.experimental.pallas import tpu_sc as plsc

B = 8
L = 1024
VOCAB = 8192
NUM_CODES = 1024
CODE_DIM = 64
EMBED_DIM = 128
HIDDEN_DIM = 256

PAD = 8
LP = L + 2 * PAD
CH = 256
NCH = L // CH

_F32 = jnp.float32
_BF16 = jnp.bfloat16


def _sc_gather_padded(table, idx, zrows):
    n, d = idx.shape[0], table.shape[1]
    info = plsc.get_sparse_core_info()
    nw = info.num_cores * info.num_subcores
    b_per_w = n // nw
    mesh = plsc.VectorSubcoreMesh(core_axis_name="c", subcore_axis_name="s")

    @functools.partial(
        pl.kernel,
        mesh=mesh,
        out_type=jax.ShapeDtypeStruct((B * LP, d), _F32),
        scratch_types=[
            pltpu.VMEM((b_per_w,), jnp.int32),
            pltpu.VMEM((b_per_w, d), _F32),
            pltpu.SemaphoreType.DMA,
        ],
    )
    def k(table_hbm, idx_hbm, zrows_hbm, out_hbm, idx_v, rows_v, sem):
        wid = lax.axis_index("s") * info.num_cores + lax.axis_index("c")
        base = wid * b_per_w
        seq = base // L
        dst = base + seq * 2 * PAD + PAD
        pltpu.sync_copy(idx_hbm.at[pl.ds(base, b_per_w)], idx_v)
        pltpu.async_copy(table_hbm.at[idx_v], rows_v, sem).wait()
        pltpu.sync_copy(rows_v, out_hbm.at[pl.ds(dst, b_per_w)])

        @pl.when(base % L == 0)
        def _top_halo():
            pltpu.sync_copy(zrows_hbm, out_hbm.at[pl.ds(dst - PAD, PAD)])

        @pl.when((base + b_per_w) % L == 0)
        def _bot_halo():
            pltpu.sync_copy(zrows_hbm, out_hbm.at[pl.ds(dst + b_per_w, PAD)])

    return k(table, idx, zrows)


def _conv3s(x, w, b, dtype):
    cin = x.shape[1]
    xc = x.astype(dtype)
    xm = jnp.concatenate([jnp.zeros((1, cin), dtype), xc[:-1, :]], axis=0)
    xp = jnp.concatenate([xc[1:, :], jnp.zeros((1, cin), dtype)], axis=0)
    acc = lax.dot(xm, w[0], preferred_element_type=_F32)
    acc += lax.dot(xc, w[1], preferred_element_type=_F32)
    acc += lax.dot(xp, w[2], preferred_element_type=_F32)
    return acc + b


def _fused_body(emb_ref, ew1, eb1, ew2, eb2, ew3, eb3, vqt, vq2, vq,
                dw1, db1, dw2, db2, dw3, db3, wt_ref, ob_ref,
                logits_ref, codes_ref, loss_ref):
    j = pl.program_id(1)
    a = j * CH

    x_t = emb_ref[0, pl.ds(a, CH + 2 * PAD), :]
    h1 = jax.nn.relu(_conv3s(x_t, ew1, eb1[...], _F32))
    rows = lax.broadcasted_iota(jnp.int32, (CH + 2 * PAD, 1), 0) + a - PAD
    h1 = jnp.where((rows >= 0) & (rows < L), h1, 0.0)
    h2 = jax.nn.relu(_conv3s(h1, ew2, eb2[...], _F32))
    z_e = lax.dot(h2[4:CH + 12], ew3[...],
                  preferred_element_type=_F32) + eb3[...]

    zsq = jnp.sum(z_e * z_e, axis=1, keepdims=True)
    dists = zsq - 2.0 * lax.dot(z_e, vqt[...],
                                preferred_element_type=_F32) + vq2[...]
    m = jnp.min(dists, axis=1, keepdims=True)
    iota = lax.broadcasted_iota(jnp.int32, (CH + PAD, NUM_CODES), 1)
    codes = jnp.min(jnp.where(dists == m, iota, NUM_CODES), axis=1)
    codes_ref[0, 0, :] = codes[4:CH + 4]

    onehot = (iota == codes[:, None]).astype(_BF16)
    z_q = lax.dot(onehot, vq[...], preferred_element_type=_F32)
    diff = z_e[4:CH + 4] - z_q[4:CH + 4]
    loss_ref[0] = jnp.sum(diff * diff, axis=0, keepdims=True)

    d1 = jax.nn.relu(
        lax.dot(z_q.astype(_BF16), dw1[...],
                preferred_element_type=_F32) + db1[...])
    rows6 = lax.broadcasted_iota(jnp.int32, (CH + PAD, 1), 0) + a - 4
    d1 = jnp.where((rows6 >= 0) & (rows6 < L), d1, 0.0)
    d2 = jax.nn.relu(_conv3s(d1, dw2, db2[...], _BF16))
    d3 = jax.nn.relu(
        lax.dot(d2[4:CH + 4].astype(_BF16), dw3[...],
                preferred_element_type=_F32) + db3[...]).astype(_BF16)

    logits_ref[0] = lax.dot(d3, wt_ref[...],
                            preferred_element_type=_F32) + ob_ref[...]


def _fused(emb_p, ew1, eb1, ew2, eb2, ew3, eb3, vqt, vq2, vq,
           dw1, db1, dw2, db2, dw3, db3, out_wt, out_b2):
    full = lambda s: pl.BlockSpec(s, lambda i, j: (0,) * len(s))
    return pl.pallas_call(
        _fused_body,
        grid=(B, NCH),
        in_specs=[
            pl.BlockSpec((1, LP, EMBED_DIM), lambda i, j: (i, 0, 0)),
            full((3, EMBED_DIM, HIDDEN_DIM)), full((1, HIDDEN_DIM)),
            full((3, HIDDEN_DIM, HIDDEN_DIM)), full((1, HIDDEN_DIM)),
            full((HIDDEN_DIM, CODE_DIM)), full((1, CODE_DIM)),
            full((CODE_DIM, NUM_CODES)), full((1, NUM_CODES)),
            full((NUM_CODES, CODE_DIM)),
            full((CODE_DIM, HIDDEN_DIM)), full((1, HIDDEN_DIM)),
            full((3, HIDDEN_DIM, HIDDEN_DIM)), full((1, HIDDEN_DIM)),
            full((HIDDEN_DIM, EMBED_DIM)), full((1, EMBED_DIM)),
            full((EMBED_DIM, VOCAB)),
            full((1, VOCAB)),
        ],
        out_specs=[
            pl.BlockSpec((1, CH, VOCAB), lambda i, j: (i, j, 0)),
            pl.BlockSpec((1, 1, CH), lambda i, j: (i * NCH + j, 0, 0)),
            pl.BlockSpec((1, 1, CODE_DIM), lambda i, j: (i * NCH + j, 0, 0)),
        ],
        out_shape=[
            jax.ShapeDtypeStruct((B, L, VOCAB), _F32),
            jax.ShapeDtypeStruct((B * NCH, 1, CH), jnp.int32),
            jax.ShapeDtypeStruct((B * NCH, 1, CODE_DIM), _F32),
        ],
    )(emb_p, ew1, eb1, ew2, eb2, ew3, eb3, vqt, vq2, vq,
      dw1, db1, dw2, db2, dw3, db3, out_wt, out_b2)


def kernel(x, token_emb, enc_w1, enc_b1, enc_w2, enc_b2, enc_w3, enc_b3,
           vq_emb, dec_w1, dec_b1, dec_w2, dec_b2, dec_w3, dec_b3,
           out_w, out_b):
    zrows = jnp.zeros((PAD, EMBED_DIM), _F32)
    emb_p = _sc_gather_padded(token_emb, x.reshape(-1).astype(jnp.int32),
                              zrows)
    emb_p = emb_p.reshape(B, LP, EMBED_DIM)

    ew1 = jnp.transpose(enc_w1, (2, 1, 0))
    ew2 = jnp.transpose(enc_w2, (2, 1, 0))
    ew3 = enc_w3[:, :, 0].T
    dw1 = dec_w1[:, :, 0].T.astype(_BF16)
    dw2 = jnp.transpose(dec_w2, (2, 1, 0)).astype(_BF16)
    dw3 = dec_w3[:, :, 0].T.astype(_BF16)
    vqt = vq_emb.T
    vq2 = jnp.sum(vq_emb * vq_emb, axis=1)[None, :]
    vqb = vq_emb.astype(_BF16)

    logits, codes3, loss_parts = _fused(
        emb_p, ew1, enc_b1[None, :], ew2, enc_b2[None, :], ew3,
        enc_b3[None, :], vqt, vq2, vqb,
        dw1, dec_b1[None, :], dw2, dec_b2[None, :], dw3, dec_b3[None, :],
        out_w.T.astype(_BF16), out_b[None, :])

    codes = codes3.reshape(B, L)
    loss_vq = 0.1 * jnp.sum(loss_parts) / (B * L * CODE_DIM)
    return logits, loss_vq, codes


# speedup vs baseline: 1.1279x; 1.1279x over previous
import functools

import jax
import jax.numpy as jnp
from jax import lax
from jax.experimental import pallas as pl
from jax.experimental.pallas import tpu as pltpu
from jax.experimental.pallas import tpu_sc as plsc

B = 8
L = 1024
VOCAB = 8192
NUM_CODES = 1024
CODE_DIM = 64
EMBED_DIM = 128
HIDDEN_DIM = 256

PAD = 8
LP = L + 2 * PAD
CH = 512
NCH = L // CH

_F32 = jnp.float32
_BF16 = jnp.bfloat16


def _sc_gather_padded(table, idx, zrows):
    n, d = idx.shape[0], table.shape[1]
    info = plsc.get_sparse_core_info()
    nw = info.num_cores * info.num_subcores
    b_per_w = n // nw
    mesh = plsc.VectorSubcoreMesh(core_axis_name="c", subcore_axis_name="s")

    @functools.partial(
        pl.kernel,
        mesh=mesh,
        out_type=jax.ShapeDtypeStruct((B * LP, d), _F32),
        scratch_types=[
            pltpu.VMEM((b_per_w,), jnp.int32),
            pltpu.VMEM((b_per_w, d), _F32),
            pltpu.SemaphoreType.DMA,
        ],
    )
    def k(table_hbm, idx_hbm, zrows_hbm, out_hbm, idx_v, rows_v, sem):
        wid = lax.axis_index("s") * info.num_cores + lax.axis_index("c")
        base = wid * b_per_w
        seq = base // L
        dst = base + seq * 2 * PAD + PAD
        pltpu.sync_copy(idx_hbm.at[pl.ds(base, b_per_w)], idx_v)
        pltpu.async_copy(table_hbm.at[idx_v], rows_v, sem).wait()
        pltpu.sync_copy(rows_v, out_hbm.at[pl.ds(dst, b_per_w)])

        @pl.when(base % L == 0)
        def _top_halo():
            pltpu.sync_copy(zrows_hbm, out_hbm.at[pl.ds(dst - PAD, PAD)])

        @pl.when((base + b_per_w) % L == 0)
        def _bot_halo():
            pltpu.sync_copy(zrows_hbm, out_hbm.at[pl.ds(dst + b_per_w, PAD)])

    return k(table, idx, zrows)


def _conv3s(x, w, b, dtype):
    cin = x.shape[1]
    xc = x.astype(dtype)
    xm = jnp.concatenate([jnp.zeros((1, cin), dtype), xc[:-1, :]], axis=0)
    xp = jnp.concatenate([xc[1:, :], jnp.zeros((1, cin), dtype)], axis=0)
    acc = lax.dot(xm, w[0], preferred_element_type=_F32)
    acc += lax.dot(xc, w[1], preferred_element_type=_F32)
    acc += lax.dot(xp, w[2], preferred_element_type=_F32)
    return acc + b


def _fused_body(emb_ref, ew1, eb1, ew2, eb2, ew3, eb3, vqt, vq2, vq,
                dw1, db1, dw2, db2, dw3, db3, wt_ref, ob_ref,
                logits_ref, codes_ref, loss_ref):
    j = pl.program_id(1)
    a = j * CH

    x_t = emb_ref[0, pl.ds(a, CH + 2 * PAD), :]
    h1 = jax.nn.relu(_conv3s(x_t, ew1, eb1[...], _F32))
    rows = lax.broadcasted_iota(jnp.int32, (CH + 2 * PAD, 1), 0) + a - PAD
    h1 = jnp.where((rows >= 0) & (rows < L), h1, 0.0)
    h2 = jax.nn.relu(_conv3s(h1, ew2, eb2[...], _F32))
    z_e = lax.dot(h2[4:CH + 12], ew3[...],
                  preferred_element_type=_F32) + eb3[...]

    zsq = jnp.sum(z_e * z_e, axis=1, keepdims=True)
    dists = zsq - 2.0 * lax.dot(z_e, vqt[...],
                                preferred_element_type=_F32) + vq2[...]
    m = jnp.min(dists, axis=1, keepdims=True)
    iota = lax.broadcasted_iota(jnp.int32, (CH + PAD, NUM_CODES), 1)
    codes = jnp.min(jnp.where(dists == m, iota, NUM_CODES), axis=1)
    codes_ref[0, 0, :] = codes[4:CH + 4]

    onehot = (iota == codes[:, None]).astype(_BF16)
    z_q = lax.dot(onehot, vq[...], preferred_element_type=_F32)
    diff = z_e[4:CH + 4] - z_q[4:CH + 4]
    loss_ref[0] = jnp.sum(diff * diff, axis=0, keepdims=True)

    d1 = jax.nn.relu(
        lax.dot(z_q.astype(_BF16), dw1[...],
                preferred_element_type=_F32) + db1[...])
    rows6 = lax.broadcasted_iota(jnp.int32, (CH + PAD, 1), 0) + a - 4
    d1 = jnp.where((rows6 >= 0) & (rows6 < L), d1, 0.0)
    d2 = jax.nn.relu(_conv3s(d1, dw2, db2[...], _BF16))
    d3 = jax.nn.relu(
        lax.dot(d2[4:CH + 4].astype(_BF16), dw3[...],
                preferred_element_type=_F32) + db3[...]).astype(_BF16)

    logits_ref[0] = lax.dot(d3, wt_ref[...],
                            preferred_element_type=_F32) + ob_ref[...]


def _fused(emb_p, ew1, eb1, ew2, eb2, ew3, eb3, vqt, vq2, vq,
           dw1, db1, dw2, db2, dw3, db3, out_wt, out_b2):
    full = lambda s: pl.BlockSpec(s, lambda i, j: (0,) * len(s))
    return pl.pallas_call(
        _fused_body,
        grid=(B, NCH),
        in_specs=[
            pl.BlockSpec((1, LP, EMBED_DIM), lambda i, j: (i, 0, 0)),
            full((3, EMBED_DIM, HIDDEN_DIM)), full((1, HIDDEN_DIM)),
            full((3, HIDDEN_DIM, HIDDEN_DIM)), full((1, HIDDEN_DIM)),
            full((HIDDEN_DIM, CODE_DIM)), full((1, CODE_DIM)),
            full((CODE_DIM, NUM_CODES)), full((1, NUM_CODES)),
            full((NUM_CODES, CODE_DIM)),
            full((CODE_DIM, HIDDEN_DIM)), full((1, HIDDEN_DIM)),
            full((3, HIDDEN_DIM, HIDDEN_DIM)), full((1, HIDDEN_DIM)),
            full((HIDDEN_DIM, EMBED_DIM)), full((1, EMBED_DIM)),
            full((EMBED_DIM, VOCAB)),
            full((1, VOCAB)),
        ],
        out_specs=[
            pl.BlockSpec((1, CH, VOCAB), lambda i, j: (i, j, 0)),
            pl.BlockSpec((1, 1, CH), lambda i, j: (i * NCH + j, 0, 0)),
            pl.BlockSpec((1, 1, CODE_DIM), lambda i, j: (i * NCH + j, 0, 0)),
        ],
        out_shape=[
            jax.ShapeDtypeStruct((B, L, VOCAB), _F32),
            jax.ShapeDtypeStruct((B * NCH, 1, CH), jnp.int32),
            jax.ShapeDtypeStruct((B * NCH, 1, CODE_DIM), _F32),
        ],
    )(emb_p, ew1, eb1, ew2, eb2, ew3, eb3, vqt, vq2, vq,
      dw1, db1, dw2, db2, dw3, db3, out_wt, out_b2)


def kernel(x, token_emb, enc_w1, enc_b1, enc_w2, enc_b2, enc_w3, enc_b3,
           vq_emb, dec_w1, dec_b1, dec_w2, dec_b2, dec_w3, dec_b3,
           out_w, out_b):
    zrows = jnp.zeros((PAD, EMBED_DIM), _F32)
    emb_p = _sc_gather_padded(token_emb, x.reshape(-1).astype(jnp.int32),
                              zrows)
    emb_p = emb_p.reshape(B, LP, EMBED_DIM)

    ew1 = jnp.transpose(enc_w1, (2, 1, 0))
    ew2 = jnp.transpose(enc_w2, (2, 1, 0))
    ew3 = enc_w3[:, :, 0].T
    dw1 = dec_w1[:, :, 0].T.astype(_BF16)
    dw2 = jnp.transpose(dec_w2, (2, 1, 0)).astype(_BF16)
    dw3 = dec_w3[:, :, 0].T.astype(_BF16)
    vqt = vq_emb.T
    vq2 = jnp.sum(vq_emb * vq_emb, axis=1)[None, :]
    vqb = vq_emb.astype(_BF16)

    logits, codes3, loss_parts = _fused(
        emb_p, ew1, enc_b1[None, :], ew2, enc_b2[None, :], ew3,
        enc_b3[None, :], vqt, vq2, vqb,
        dw1, dec_b1[None, :], dw2, dec_b2[None, :], dw3, dec_b3[None, :],
        out_w.T.astype(_BF16), out_b[None, :])

    codes = codes3.reshape(B, L)
    loss_vq = 0.1 * jnp.sum(loss_parts) / (B * L * CODE_DIM)
    return logits, loss_vq, codes


# speedup vs baseline: 1.1297x; 1.0016x over previous
import functools

import jax
import jax.numpy as jnp
from jax import lax
from jax.experimental import pallas as pl
from jax.experimental.pallas import tpu as pltpu
from jax.experimental.pallas import tpu_sc as plsc

B = 8
L = 1024
VOCAB = 8192
NUM_CODES = 1024
CODE_DIM = 64
EMBED_DIM = 128
HIDDEN_DIM = 256

PAD = 8
LP = L + 2 * PAD
CH = 512
NCH = L // CH

_F32 = jnp.float32
_BF16 = jnp.bfloat16


def _sc_gather_padded(table, idx):
    n, d = idx.shape[0], table.shape[1]
    info = plsc.get_sparse_core_info()
    nw = info.num_cores * info.num_subcores
    b_per_w = n // nw
    mesh = plsc.VectorSubcoreMesh(core_axis_name="c", subcore_axis_name="s")

    @functools.partial(
        pl.kernel,
        mesh=mesh,
        out_type=jax.ShapeDtypeStruct((B * LP, d), _F32),
        scratch_types=[
            pltpu.VMEM((b_per_w,), jnp.int32),
            pltpu.VMEM((b_per_w, d), _F32),
            pltpu.SemaphoreType.DMA,
        ],
    )
    def k(table_hbm, idx_hbm, out_hbm, idx_v, rows_v, sem):
        wid = lax.axis_index("s") * info.num_cores + lax.axis_index("c")
        base = wid * b_per_w
        seq = base // L
        dst = base + seq * 2 * PAD + PAD
        pltpu.sync_copy(idx_hbm.at[pl.ds(base, b_per_w)], idx_v)
        pltpu.async_copy(table_hbm.at[idx_v], rows_v, sem).wait()
        pltpu.sync_copy(rows_v, out_hbm.at[pl.ds(dst, b_per_w)])

    return k(table, idx)


def _conv3s(x, w, b, dtype):
    cin = x.shape[1]
    xc = x.astype(dtype)
    xm = jnp.concatenate([jnp.zeros((1, cin), dtype), xc[:-1, :]], axis=0)
    xp = jnp.concatenate([xc[1:, :], jnp.zeros((1, cin), dtype)], axis=0)
    acc = lax.dot(xm, w[0], preferred_element_type=_F32)
    acc += lax.dot(xc, w[1], preferred_element_type=_F32)
    acc += lax.dot(xp, w[2], preferred_element_type=_F32)
    return acc + b


def _fused_body(emb_ref, ew1, eb1, ew2, eb2, ew3, eb3, vqt, vq2, vq,
                dw1, db1, dw2, db2, dw3, db3, wt_ref, ob_ref,
                logits_ref, codes_ref, loss_ref):
    j = pl.program_id(1)
    a = j * CH

    x_t = emb_ref[0, pl.ds(a, CH + 2 * PAD), :]
    rows = lax.broadcasted_iota(jnp.int32, (CH + 2 * PAD, 1), 0) + a - PAD
    in_seq = (rows >= 0) & (rows < L)
    x_t = jnp.where(in_seq, x_t, 0.0)
    h1 = jax.nn.relu(_conv3s(x_t, ew1, eb1[...], _F32))
    h1 = jnp.where(in_seq, h1, 0.0)
    h2 = jax.nn.relu(_conv3s(h1, ew2, eb2[...], _F32))
    z_e = lax.dot(h2[4:CH + 12], ew3[...],
                  preferred_element_type=_F32) + eb3[...]

    zsq = jnp.sum(z_e * z_e, axis=1, keepdims=True)
    dists = zsq - 2.0 * lax.dot(z_e, vqt[...],
                                preferred_element_type=_F32) + vq2[...]
    m = jnp.min(dists, axis=1, keepdims=True)
    iota = lax.broadcasted_iota(jnp.int32, (CH + PAD, NUM_CODES), 1)
    codes = jnp.min(jnp.where(dists == m, iota, NUM_CODES), axis=1)
    codes_ref[0, 0, :] = codes[4:CH + 4]

    onehot = (iota == codes[:, None]).astype(_BF16)
    z_q = lax.dot(onehot, vq[...], preferred_element_type=_F32)
    diff = z_e[4:CH + 4] - z_q[4:CH + 4]
    loss_ref[0] = jnp.sum(diff * diff, axis=0, keepdims=True)

    d1 = jax.nn.relu(
        lax.dot(z_q.astype(_BF16), dw1[...],
                preferred_element_type=_F32) + db1[...])
    rows6 = lax.broadcasted_iota(jnp.int32, (CH + PAD, 1), 0) + a - 4
    d1 = jnp.where((rows6 >= 0) & (rows6 < L), d1, 0.0)
    d2 = jax.nn.relu(_conv3s(d1, dw2, db2[...], _BF16))
    d3 = jax.nn.relu(
        lax.dot(d2[4:CH + 4].astype(_BF16), dw3[...],
                preferred_element_type=_F32) + db3[...]).astype(_BF16)

    logits_ref[0] = lax.dot(d3, wt_ref[...],
                            preferred_element_type=_F32) + ob_ref[...]


def _fused(emb_p, ew1, eb1, ew2, eb2, ew3, eb3, vqt, vq2, vq,
           dw1, db1, dw2, db2, dw3, db3, out_wt, out_b2):
    full = lambda s: pl.BlockSpec(s, lambda i, j: (0,) * len(s))
    return pl.pallas_call(
        _fused_body,
        grid=(B, NCH),
        in_specs=[
            pl.BlockSpec((1, LP, EMBED_DIM), lambda i, j: (i, 0, 0)),
            full((3, EMBED_DIM, HIDDEN_DIM)), full((1, HIDDEN_DIM)),
            full((3, HIDDEN_DIM, HIDDEN_DIM)), full((1, HIDDEN_DIM)),
            full((HIDDEN_DIM, CODE_DIM)), full((1, CODE_DIM)),
            full((CODE_DIM, NUM_CODES)), full((1, NUM_CODES)),
            full((NUM_CODES, CODE_DIM)),
            full((CODE_DIM, HIDDEN_DIM)), full((1, HIDDEN_DIM)),
            full((3, HIDDEN_DIM, HIDDEN_DIM)), full((1, HIDDEN_DIM)),
            full((HIDDEN_DIM, EMBED_DIM)), full((1, EMBED_DIM)),
            full((EMBED_DIM, VOCAB)),
            full((1, VOCAB)),
        ],
        out_specs=[
            pl.BlockSpec((1, CH, VOCAB), lambda i, j: (i, j, 0)),
            pl.BlockSpec((1, 1, CH), lambda i, j: (i * NCH + j, 0, 0)),
            pl.BlockSpec((1, 1, CODE_DIM), lambda i, j: (i * NCH + j, 0, 0)),
        ],
        out_shape=[
            jax.ShapeDtypeStruct((B, L, VOCAB), _F32),
            jax.ShapeDtypeStruct((B * NCH, 1, CH), jnp.int32),
            jax.ShapeDtypeStruct((B * NCH, 1, CODE_DIM), _F32),
        ],
    )(emb_p, ew1, eb1, ew2, eb2, ew3, eb3, vqt, vq2, vq,
      dw1, db1, dw2, db2, dw3, db3, out_wt, out_b2)


def kernel(x, token_emb, enc_w1, enc_b1, enc_w2, enc_b2, enc_w3, enc_b3,
           vq_emb, dec_w1, dec_b1, dec_w2, dec_b2, dec_w3, dec_b3,
           out_w, out_b):
    emb_p = _sc_gather_padded(token_emb, x.reshape(-1).astype(jnp.int32))
    emb_p = emb_p.reshape(B, LP, EMBED_DIM)

    ew1 = jnp.transpose(enc_w1, (2, 1, 0))
    ew2 = jnp.transpose(enc_w2, (2, 1, 0))
    ew3 = enc_w3[:, :, 0].T
    dw1 = dec_w1[:, :, 0].T.astype(_BF16)
    dw2 = jnp.transpose(dec_w2, (2, 1, 0)).astype(_BF16)
    dw3 = dec_w3[:, :, 0].T.astype(_BF16)
    vqt = vq_emb.T
    vq2 = jnp.sum(vq_emb * vq_emb, axis=1)[None, :]
    vqb = vq_emb.astype(_BF16)

    logits, codes3, loss_parts = _fused(
        emb_p, ew1, enc_b1[None, :], ew2, enc_b2[None, :], ew3,
        enc_b3[None, :], vqt, vq2, vqb,
        dw1, dec_b1[None, :], dw2, dec_b2[None, :], dw3, dec_b3[None, :],
        out_w.T.astype(_BF16), out_b[None, :])

    codes = codes3.reshape(B, L)
    loss_vq = 0.1 * jnp.sum(loss_parts) / (B * L * CODE_DIM)
    return logits, loss_vq, codes


# speedup vs baseline: 1.1709x; 1.0364x over previous
import functools

import jax
import jax.numpy as jnp
from jax import lax
from jax.experimental import pallas as pl
from jax.experimental.pallas import tpu as pltpu
from jax.experimental.pallas import tpu_sc as plsc

B = 8
L = 1024
VOCAB = 8192
NUM_CODES = 1024
CODE_DIM = 64
EMBED_DIM = 128
HIDDEN_DIM = 256

PAD = 8
LP = L + 2 * PAD
CH = 512
NCH = L // CH

_F32 = jnp.float32
_BF16 = jnp.bfloat16


def _sc_gather_padded(table, idx):
    n, d = idx.shape[0], table.shape[1]
    info = plsc.get_sparse_core_info()
    nw = info.num_cores * info.num_subcores
    b_per_w = n // nw
    mesh = plsc.VectorSubcoreMesh(core_axis_name="c", subcore_axis_name="s")

    @functools.partial(
        pl.kernel,
        mesh=mesh,
        out_type=jax.ShapeDtypeStruct((B * LP, d), _F32),
        scratch_types=[
            pltpu.VMEM((b_per_w,), jnp.int32),
            pltpu.VMEM((b_per_w, d), _F32),
            pltpu.SemaphoreType.DMA,
        ],
    )
    def k(table_hbm, idx_hbm, out_hbm, idx_v, rows_v, sem):
        wid = lax.axis_index("s") * info.num_cores + lax.axis_index("c")
        base = wid * b_per_w
        seq = base // L
        dst = base + seq * 2 * PAD + PAD
        pltpu.sync_copy(idx_hbm.at[pl.ds(base, b_per_w)], idx_v)
        pltpu.async_copy(table_hbm.at[idx_v], rows_v, sem).wait()
        pltpu.sync_copy(rows_v, out_hbm.at[pl.ds(dst, b_per_w)])

    return k(table, idx)


def _conv3s(x, w, b, dtype):
    cin = x.shape[1]
    xc = x.astype(dtype)
    xm = jnp.concatenate([jnp.zeros((1, cin), dtype), xc[:-1, :]], axis=0)
    xp = jnp.concatenate([xc[1:, :], jnp.zeros((1, cin), dtype)], axis=0)
    acc = lax.dot(xm, w[0], preferred_element_type=_F32)
    acc += lax.dot(xc, w[1], preferred_element_type=_F32)
    acc += lax.dot(xp, w[2], preferred_element_type=_F32)
    return acc + b


def _fused_body(emb_ref, ew1, eb1, ew2, eb2, ew3, eb3, vqt, vq2, vq,
                dw1, db1, dw2, db2, dw3, db3, wt_ref, ob_ref,
                logits_ref, codes_ref, loss_ref):
    j = pl.program_id(1)
    a = j * CH

    x_t = emb_ref[0, pl.ds(a, CH + 2 * PAD), :]
    rows = lax.broadcasted_iota(jnp.int32, (CH + 2 * PAD, 1), 0) + a - PAD
    in_seq = (rows >= 0) & (rows < L)
    x_t = jnp.where(in_seq, x_t, 0.0)
    h1 = jax.nn.relu(_conv3s(x_t, ew1, eb1[...], _F32))
    h1 = jnp.where(in_seq, h1, 0.0)
    h2 = jax.nn.relu(_conv3s(h1, ew2, eb2[...], _F32))
    z_e = lax.dot(h2[4:CH + 12], ew3[...],
                  preferred_element_type=_F32) + eb3[...]

    zsq = jnp.sum(z_e * z_e, axis=1, keepdims=True)
    dists = zsq - 2.0 * lax.dot(z_e, vqt[...],
                                preferred_element_type=_F32) + vq2[...]
    m = jnp.min(dists, axis=1, keepdims=True)
    iota = lax.broadcasted_iota(jnp.int32, (CH + PAD, NUM_CODES), 1)
    codes = jnp.min(jnp.where(dists == m, iota, NUM_CODES), axis=1)
    codes_ref[0, 0, :] = codes[4:CH + 4]

    onehot = (iota == codes[:, None]).astype(_BF16)
    z_q = lax.dot(onehot, vq[...], preferred_element_type=_F32)
    diff = z_e[4:CH + 4] - z_q[4:CH + 4]
    loss_ref[0] = jnp.sum(diff * diff, axis=0, keepdims=True)

    d1 = jax.nn.relu(
        lax.dot(z_q.astype(_BF16), dw1[...],
                preferred_element_type=_F32) + db1[...])
    rows6 = lax.broadcasted_iota(jnp.int32, (CH + PAD, 1), 0) + a - 4
    d1 = jnp.where((rows6 >= 0) & (rows6 < L), d1, 0.0)
    d2 = jax.nn.relu(_conv3s(d1, dw2, db2[...], _BF16))
    d3 = jax.nn.relu(
        lax.dot(d2[4:CH + 4].astype(_BF16), dw3[...],
                preferred_element_type=_F32) + db3[...]).astype(_BF16)

    logits_ref[0] = lax.dot_general(
        d3, wt_ref[...].astype(_BF16), (((1,), (1,)), ((), ())),
        preferred_element_type=_F32) + ob_ref[...]


def _fused(emb_p, ew1, eb1, ew2, eb2, ew3, eb3, vqt, vq2, vq,
           dw1, db1, dw2, db2, dw3, db3, out_wt, out_b2):
    full = lambda s: pl.BlockSpec(s, lambda i, j: (0,) * len(s))
    return pl.pallas_call(
        _fused_body,
        grid=(B, NCH),
        in_specs=[
            pl.BlockSpec((1, LP, EMBED_DIM), lambda i, j: (i, 0, 0)),
            full((3, EMBED_DIM, HIDDEN_DIM)), full((1, HIDDEN_DIM)),
            full((3, HIDDEN_DIM, HIDDEN_DIM)), full((1, HIDDEN_DIM)),
            full((HIDDEN_DIM, CODE_DIM)), full((1, CODE_DIM)),
            full((CODE_DIM, NUM_CODES)), full((1, NUM_CODES)),
            full((NUM_CODES, CODE_DIM)),
            full((CODE_DIM, HIDDEN_DIM)), full((1, HIDDEN_DIM)),
            full((3, HIDDEN_DIM, HIDDEN_DIM)), full((1, HIDDEN_DIM)),
            full((HIDDEN_DIM, EMBED_DIM)), full((1, EMBED_DIM)),
            full((VOCAB, EMBED_DIM)),
            full((1, VOCAB)),
        ],
        out_specs=[
            pl.BlockSpec((1, CH, VOCAB), lambda i, j: (i, j, 0)),
            pl.BlockSpec((1, 1, CH), lambda i, j: (i * NCH + j, 0, 0)),
            pl.BlockSpec((1, 1, CODE_DIM), lambda i, j: (i * NCH + j, 0, 0)),
        ],
        out_shape=[
            jax.ShapeDtypeStruct((B, L, VOCAB), _F32),
            jax.ShapeDtypeStruct((B * NCH, 1, CH), jnp.int32),
            jax.ShapeDtypeStruct((B * NCH, 1, CODE_DIM), _F32),
        ],
    )(emb_p, ew1, eb1, ew2, eb2, ew3, eb3, vqt, vq2, vq,
      dw1, db1, dw2, db2, dw3, db3, out_wt, out_b2)


def kernel(x, token_emb, enc_w1, enc_b1, enc_w2, enc_b2, enc_w3, enc_b3,
           vq_emb, dec_w1, dec_b1, dec_w2, dec_b2, dec_w3, dec_b3,
           out_w, out_b):
    emb_p = _sc_gather_padded(token_emb, x.reshape(-1).astype(jnp.int32))
    emb_p = emb_p.reshape(B, LP, EMBED_DIM)

    ew1 = jnp.transpose(enc_w1, (2, 1, 0))
    ew2 = jnp.transpose(enc_w2, (2, 1, 0))
    ew3 = enc_w3[:, :, 0].T
    dw1 = dec_w1[:, :, 0].T.astype(_BF16)
    dw2 = jnp.transpose(dec_w2, (2, 1, 0)).astype(_BF16)
    dw3 = dec_w3[:, :, 0].T.astype(_BF16)
    vqt = vq_emb.T
    vq2 = jnp.sum(vq_emb * vq_emb, axis=1)[None, :]
    vqb = vq_emb.astype(_BF16)

    logits, codes3, loss_parts = _fused(
        emb_p, ew1, enc_b1[None, :], ew2, enc_b2[None, :], ew3,
        enc_b3[None, :], vqt, vq2, vqb,
        dw1, dec_b1[None, :], dw2, dec_b2[None, :], dw3, dec_b3[None, :],
        out_w, out_b[None, :])

    codes = codes3.reshape(B, L)
    loss_vq = 0.1 * jnp.sum(loss_parts) / (B * L * CODE_DIM)
    return logits, loss_vq, codes


# speedup vs baseline: 1.1767x; 1.0050x over previous
import functools

import jax
import jax.numpy as jnp
from jax import lax
from jax.experimental import pallas as pl
from jax.experimental.pallas import tpu as pltpu
from jax.experimental.pallas import tpu_sc as plsc

B = 8
L = 1024
VOCAB = 8192
NUM_CODES = 1024
CODE_DIM = 64
EMBED_DIM = 128
HIDDEN_DIM = 256

PAD = 8
LP = L + 2 * PAD
CH = 512
NCH = L // CH

_F32 = jnp.float32
_BF16 = jnp.bfloat16


def _sc_gather_padded(table, idx):
    n, d = idx.shape[0], table.shape[1]
    info = plsc.get_sparse_core_info()
    nw = info.num_cores * info.num_subcores
    b_per_w = n // nw
    mesh = plsc.VectorSubcoreMesh(core_axis_name="c", subcore_axis_name="s")

    @functools.partial(
        pl.kernel,
        mesh=mesh,
        out_type=jax.ShapeDtypeStruct((B * LP, d), _F32),
        scratch_types=[
            pltpu.VMEM((b_per_w,), jnp.int32),
            pltpu.VMEM((b_per_w, d), _F32),
            pltpu.SemaphoreType.DMA,
        ],
    )
    def k(table_hbm, idx_hbm, out_hbm, idx_v, rows_v, sem):
        wid = lax.axis_index("s") * info.num_cores + lax.axis_index("c")
        base = wid * b_per_w
        seq = base // L
        dst = base + seq * 2 * PAD + PAD
        pltpu.sync_copy(idx_hbm.at[pl.ds(base, b_per_w)], idx_v)
        pltpu.async_copy(table_hbm.at[idx_v], rows_v, sem).wait()
        pltpu.sync_copy(rows_v, out_hbm.at[pl.ds(dst, b_per_w)])

    return k(table, idx)


def _conv3s(x, w, b, dtype):
    cin = x.shape[1]
    xc = x.astype(dtype)
    xm = jnp.concatenate([jnp.zeros((1, cin), dtype), xc[:-1, :]], axis=0)
    xp = jnp.concatenate([xc[1:, :], jnp.zeros((1, cin), dtype)], axis=0)
    xcat = jnp.concatenate([xm, xc, xp], axis=1)
    wcat = w[...].reshape(3 * cin, w.shape[2]).astype(dtype)
    return lax.dot(xcat, wcat, preferred_element_type=_F32) + b


def _fused_body(emb_ref, ew1, eb1, ew2, eb2, ew3, eb3, vqt, vq2, vq,
                dw1, db1, dw2, db2, dw3, db3, wt_ref, ob_ref,
                logits_ref, codes_ref, loss_ref):
    j = pl.program_id(1)
    a = j * CH

    x_t = emb_ref[0, pl.ds(a, CH + 2 * PAD), :]
    rows = lax.broadcasted_iota(jnp.int32, (CH + 2 * PAD, 1), 0) + a - PAD
    in_seq = (rows >= 0) & (rows < L)
    x_t = jnp.where(in_seq, x_t, 0.0)
    h1 = jax.nn.relu(_conv3s(x_t, ew1, eb1[...], _F32))
    h1 = jnp.where(in_seq, h1, 0.0)
    h2 = jax.nn.relu(_conv3s(h1, ew2, eb2[...], _F32))
    z_e = lax.dot(h2[4:CH + 12], ew3[...],
                  preferred_element_type=_F32) + eb3[...]

    zsq = jnp.sum(z_e * z_e, axis=1, keepdims=True)
    dists = zsq - 2.0 * lax.dot(z_e, vqt[...],
                                preferred_element_type=_F32) + vq2[...]
    m = jnp.min(dists, axis=1, keepdims=True)
    iota = lax.broadcasted_iota(jnp.int32, (CH + PAD, NUM_CODES), 1)
    codes = jnp.min(jnp.where(dists == m, iota, NUM_CODES), axis=1)
    codes_ref[0, 0, :] = codes[4:CH + 4]

    onehot = (iota == codes[:, None]).astype(_BF16)
    z_q = lax.dot(onehot, vq[...], preferred_element_type=_F32)
    diff = z_e[4:CH + 4] - z_q[4:CH + 4]
    loss_ref[0] = jnp.sum(diff * diff, axis=0, keepdims=True)

    d1 = jax.nn.relu(
        lax.dot(z_q.astype(_BF16), dw1[...],
                preferred_element_type=_F32) + db1[...])
    rows6 = lax.broadcasted_iota(jnp.int32, (CH + PAD, 1), 0) + a - 4
    d1 = jnp.where((rows6 >= 0) & (rows6 < L), d1, 0.0)
    d2 = jax.nn.relu(_conv3s(d1, dw2, db2[...], _BF16))
    d3 = jax.nn.relu(
        lax.dot(d2[4:CH + 4].astype(_BF16), dw3[...],
                preferred_element_type=_F32) + db3[...]).astype(_BF16)

    logits_ref[0] = lax.dot_general(
        d3, wt_ref[...].astype(_BF16), (((1,), (1,)), ((), ())),
        preferred_element_type=_F32) + ob_ref[...]


def _fused(emb_p, ew1, eb1, ew2, eb2, ew3, eb3, vqt, vq2, vq,
           dw1, db1, dw2, db2, dw3, db3, out_wt, out_b2):
    full = lambda s: pl.BlockSpec(s, lambda i, j: (0,) * len(s))
    return pl.pallas_call(
        _fused_body,
        grid=(B, NCH),
        in_specs=[
            pl.BlockSpec((1, LP, EMBED_DIM), lambda i, j: (i, 0, 0)),
            full((3, EMBED_DIM, HIDDEN_DIM)), full((1, HIDDEN_DIM)),
            full((3, HIDDEN_DIM, HIDDEN_DIM)), full((1, HIDDEN_DIM)),
            full((HIDDEN_DIM, CODE_DIM)), full((1, CODE_DIM)),
            full((CODE_DIM, NUM_CODES)), full((1, NUM_CODES)),
            full((NUM_CODES, CODE_DIM)),
            full((CODE_DIM, HIDDEN_DIM)), full((1, HIDDEN_DIM)),
            full((3, HIDDEN_DIM, HIDDEN_DIM)), full((1, HIDDEN_DIM)),
            full((HIDDEN_DIM, EMBED_DIM)), full((1, EMBED_DIM)),
            full((VOCAB, EMBED_DIM)),
            full((1, VOCAB)),
        ],
        out_specs=[
            pl.BlockSpec((1, CH, VOCAB), lambda i, j: (i, j, 0)),
            pl.BlockSpec((1, 1, CH), lambda i, j: (i * NCH + j, 0, 0)),
            pl.BlockSpec((1, 1, CODE_DIM), lambda i, j: (i * NCH + j, 0, 0)),
        ],
        out_shape=[
            jax.ShapeDtypeStruct((B, L, VOCAB), _F32),
            jax.ShapeDtypeStruct((B * NCH, 1, CH), jnp.int32),
            jax.ShapeDtypeStruct((B * NCH, 1, CODE_DIM), _F32),
        ],
    )(emb_p, ew1, eb1, ew2, eb2, ew3, eb3, vqt, vq2, vq,
      dw1, db1, dw2, db2, dw3, db3, out_wt, out_b2)


def kernel(x, token_emb, enc_w1, enc_b1, enc_w2, enc_b2, enc_w3, enc_b3,
           vq_emb, dec_w1, dec_b1, dec_w2, dec_b2, dec_w3, dec_b3,
           out_w, out_b):
    emb_p = _sc_gather_padded(token_emb, x.reshape(-1).astype(jnp.int32))
    emb_p = emb_p.reshape(B, LP, EMBED_DIM)

    ew1 = jnp.transpose(enc_w1, (2, 1, 0))
    ew2 = jnp.transpose(enc_w2, (2, 1, 0))
    ew3 = enc_w3[:, :, 0].T
    dw1 = dec_w1[:, :, 0].T.astype(_BF16)
    dw2 = jnp.transpose(dec_w2, (2, 1, 0)).astype(_BF16)
    dw3 = dec_w3[:, :, 0].T.astype(_BF16)
    vqt = vq_emb.T
    vq2 = jnp.sum(vq_emb * vq_emb, axis=1)[None, :]
    vqb = vq_emb.astype(_BF16)

    logits, codes3, loss_parts = _fused(
        emb_p, ew1, enc_b1[None, :], ew2, enc_b2[None, :], ew3,
        enc_b3[None, :], vqt, vq2, vqb,
        dw1, dec_b1[None, :], dw2, dec_b2[None, :], dw3, dec_b3[None, :],
        out_w, out_b[None, :])

    codes = codes3.reshape(B, L)
    loss_vq = 0.1 * jnp.sum(loss_parts) / (B * L * CODE_DIM)
    return logits, loss_vq, codes
